# baseline (speedup 1.0000x reference)
import functools

import jax
import jax.numpy as jnp
from jax import lax
from jax.experimental import pallas as pl
from jax.experimental.pallas import tpu as pltpu
from jax.experimental.pallas import tpu_sc as plsc

VOCAB = 1000000
DIM = 64
B, T = 1024, 50
L, W = 1000, 10
LF = L * W
LPAD = 10240
TP = 56
EPS = 1e-8

NC, NS = 2, 16
NW = NC * NS
QPW = B // NW
LPW = LPAD // NW
LCH = 64
NBUF = 4


def _sc_body(qidx_hbm, lidx_hbm, table_hbm, qsum_hbm, le_hbm,
             qidx_v, lidx_v, qblk, lbuf, rows, qsems, lsem):
    c = lax.axis_index("c")
    s = lax.axis_index("s")
    wid = c * NS + s

    pltpu.sync_copy(qidx_hbm.at[pl.ds(wid * (QPW * TP), QPW * TP)], qidx_v)
    pltpu.sync_copy(lidx_hbm.at[pl.ds(wid * LPW, LPW)], lidx_v)

    for ci in range(LPW // LCH):
        pltpu.make_async_copy(
            table_hbm.at[lidx_v.at[pl.ds(ci * LCH, LCH)]],
            lbuf.at[pl.ds(ci * LCH, LCH)], lsem).start()

    def qcopy(q, b):
        return pltpu.make_async_copy(
            table_hbm.at[qidx_v.at[pl.ds(q * TP, T)]], rows[b], qsems[b])

    def accum(q, b):
        def rbody(r, accs):
            return tuple(a + rows[b][r, pl.ds(16 * d, 16)]
                         for d, a in enumerate(accs))
        accs = lax.fori_loop(
            0, T, rbody, tuple(jnp.zeros((16,), jnp.float32) for _ in range(4)))
        for d in range(4):
            qblk[q, pl.ds(16 * d, 16)] = accs[d]

    for b in range(NBUF):
        qcopy(b, b).start()
    for q in range(QPW):
        b = q % NBUF
        qcopy(q, b).wait()
        accum(q, b)
        if q + NBUF < QPW:
            qcopy(q + NBUF, b).start()

    pltpu.sync_copy(qblk, qsum_hbm.at[pl.ds(wid * QPW, QPW)])

    for ci in range(LPW // LCH):
        pltpu.make_async_copy(
            table_hbm.at[lidx_v.at[pl.ds(ci * LCH, LCH)]],
            lbuf.at[pl.ds(ci * LCH, LCH)], lsem).wait()
    pltpu.sync_copy(lbuf, le_hbm.at[pl.ds(wid * LPW, LPW)])


@functools.cache
def _sc_gather():
    return pl.kernel(
        _sc_body,
        out_type=[jax.ShapeDtypeStruct((B, DIM), jnp.float32),
                  jax.ShapeDtypeStruct((LPAD, DIM), jnp.float32)],
        mesh=plsc.VectorSubcoreMesh(core_axis_name="c", subcore_axis_name="s",
                                    num_cores=NC, num_subcores=NS),
        compiler_params=pltpu.CompilerParams(use_tc_tiling_on_sc=False),
        scratch_types=[
            pltpu.VMEM((QPW * TP,), jnp.int32),
            pltpu.VMEM((LPW,), jnp.int32),
            pltpu.VMEM((QPW, DIM), jnp.float32),
            pltpu.VMEM((LPW, DIM), jnp.float32),
            [pltpu.VMEM((T, DIM), jnp.float32) for _ in range(NBUF)],
            [pltpu.SemaphoreType.DMA for _ in range(NBUF)],
            pltpu.SemaphoreType.DMA,
        ],
    )


LBLK = 512
NLB = LPAD // LBLK


def _tc_body(qsum_ref, qidx_ref, qidxt_ref, le_ref, lab_ref, sim_ref, msk_ref):
    nw = jnp.sum((qidx_ref[...] != 0).astype(jnp.float32), axis=1,
                 keepdims=True)
    qe = qsum_ref[...] / nw
    na = jnp.maximum(jnp.sqrt(jnp.sum(qe * qe, axis=1, keepdims=True)), EPS)
    qn = qe / na
    hq = jnp.sum((qidxt_ref[...] != 0).astype(jnp.int32), axis=0,
                 keepdims=True) > 0
    le = le_ref[...]
    nb = jnp.maximum(jnp.sqrt(jnp.sum(le * le, axis=1, keepdims=True)), EPS)
    ln = le / nb
    sim_ref[...] = lax.dot_general(ln, qn, (((1,), (1,)), ((), ())),
                                   preferred_element_type=jnp.float32)
    msk_ref[...] = ((lab_ref[...].reshape(LBLK, 1) != 0) & hq).astype(jnp.int8)


def _tc_cosine(qsum, qidx, qidxt, le, lab3d):
    return pl.pallas_call(
        _tc_body,
        grid=(NLB,),
        in_specs=[
            pl.BlockSpec((B, DIM), lambda i: (0, 0)),
            pl.BlockSpec((B, TP), lambda i: (0, 0)),
            pl.BlockSpec((TP, B), lambda i: (0, 0)),
            pl.BlockSpec((LBLK, DIM), lambda i: (i, 0)),
            pl.BlockSpec((1, LBLK, 1), lambda i: (i, 0, 0)),
        ],
        out_specs=[
            pl.BlockSpec((LBLK, B), lambda i: (i, 0)),
            pl.BlockSpec((LBLK, B), lambda i: (i, 0)),
        ],
        out_shape=[jax.ShapeDtypeStruct((LF, B), jnp.float32),
                   jax.ShapeDtypeStruct((LF, B), jnp.int8)],
    )(qsum, qidx, qidxt, le, lab3d)


def kernel(queries, labels, table):
    qpad = jnp.pad(queries, ((0, 0), (0, TP - T)))
    lflat = jnp.pad(labels.T.reshape(-1), (0, LPAD - LF))
    pmat = jnp.eye(DIM, 2 * DIM, dtype=jnp.float32)
    tablep = lax.dot_general(table, pmat, (((1,), (0,)), ((), ())),
                             preferred_element_type=jnp.float32)
    tablep = tablep.reshape(2 * VOCAB, DIM)
    qsum, le = _sc_gather()(qpad.reshape(-1) * 2, lflat * 2, tablep)
    simt, maskt = _tc_cosine(qsum, qpad, qpad.T, le,
                             lflat.reshape(NLB, LBLK, 1))
    sim = simt.reshape(1, W, L, B).transpose(0, 3, 2, 1)
    msk = maskt.view(jnp.bool_)
    msk = msk.reshape(1, W, L, B).transpose(0, 3, 2, 1)
    return (sim, msk)


# speedup vs baseline: 1.0266x; 1.0266x over previous
import functools

import jax
import jax.numpy as jnp
from jax import lax
from jax.experimental import pallas as pl
from jax.experimental.pallas import tpu as pltpu
from jax.experimental.pallas import tpu_sc as plsc

VOCAB = 1000000
DIM = 64
B, T = 1024, 50
L, W = 1000, 10
LF = L * W
LPAD = 10240
TP = 56
EPS = 1e-8

NC, NS = 2, 16
NW = NC * NS
QPW = B // NW
LPW = LPAD // NW
LCH = 64
NBUF = 4


def _sc_body(qidx_hbm, lidx_hbm, table_hbm, qsum_hbm, le_hbm,
             qidx_v, lidx_v, qblk, lbuf, rows, qsems, lsem):
    c = lax.axis_index("c")
    s = lax.axis_index("s")
    wid = c * NS + s

    pltpu.sync_copy(qidx_hbm.at[pl.ds(wid * (QPW * TP), QPW * TP)], qidx_v)
    pltpu.sync_copy(lidx_hbm.at[pl.ds(wid * LPW, LPW)], lidx_v)

    for ci in range(LPW // LCH):
        pltpu.make_async_copy(
            table_hbm.at[lidx_v.at[pl.ds(ci * LCH, LCH)]],
            lbuf.at[pl.ds(ci * LCH, LCH)], lsem).start()

    def qcopy(q, b):
        return pltpu.make_async_copy(
            table_hbm.at[qidx_v.at[pl.ds(q * TP, T)]], rows[b], qsems[b])

    def accum(q, b):
        def rbody(r, accs):
            return tuple(a + rows[b][r, pl.ds(16 * d, 16)]
                         for d, a in enumerate(accs))
        accs = lax.fori_loop(
            0, T, rbody, tuple(jnp.zeros((16,), jnp.float32) for _ in range(4)))
        for d in range(4):
            qblk[q, pl.ds(16 * d, 16)] = accs[d]

    for b in range(NBUF):
        qcopy(b, b).start()
    for q in range(QPW):
        b = q % NBUF
        qcopy(q, b).wait()
        accum(q, b)
        if q + NBUF < QPW:
            qcopy(q + NBUF, b).start()

    pltpu.sync_copy(qblk, qsum_hbm.at[pl.ds(wid * QPW, QPW)])

    for ci in range(LPW // LCH):
        pltpu.make_async_copy(
            table_hbm.at[lidx_v.at[pl.ds(ci * LCH, LCH)]],
            lbuf.at[pl.ds(ci * LCH, LCH)], lsem).wait()
    pltpu.sync_copy(lbuf, le_hbm.at[pl.ds(wid * LPW, LPW)])


@functools.cache
def _sc_gather():
    return pl.kernel(
        _sc_body,
        out_type=[jax.ShapeDtypeStruct((B, DIM), jnp.float32),
                  jax.ShapeDtypeStruct((LPAD, DIM), jnp.float32)],
        mesh=plsc.VectorSubcoreMesh(core_axis_name="c", subcore_axis_name="s",
                                    num_cores=NC, num_subcores=NS),
        compiler_params=pltpu.CompilerParams(use_tc_tiling_on_sc=False),
        scratch_types=[
            pltpu.VMEM((QPW * TP,), jnp.int32),
            pltpu.VMEM((LPW,), jnp.int32),
            pltpu.VMEM((QPW, DIM), jnp.float32),
            pltpu.VMEM((LPW, DIM), jnp.float32),
            [pltpu.VMEM((T, DIM), jnp.float32) for _ in range(NBUF)],
            [pltpu.SemaphoreType.DMA for _ in range(NBUF)],
            pltpu.SemaphoreType.DMA,
        ],
    )


LBLK = 2048
NLB = LPAD // LBLK


def _tc_body(qsum_ref, qidx_ref, qidxt_ref, le_ref, lab_ref, sim_ref, msk_ref):
    nw = jnp.sum((qidx_ref[...] != 0).astype(jnp.float32), axis=1,
                 keepdims=True)
    qe = qsum_ref[...] / nw
    na = jnp.maximum(jnp.sqrt(jnp.sum(qe * qe, axis=1, keepdims=True)), EPS)
    qn = qe / na
    hq = jnp.sum((qidxt_ref[...] != 0).astype(jnp.int32), axis=0,
                 keepdims=True) > 0
    le = le_ref[...]
    nb = jnp.maximum(jnp.sqrt(jnp.sum(le * le, axis=1, keepdims=True)), EPS)
    ln = le / nb
    sim_ref[...] = lax.dot_general(ln, qn, (((1,), (1,)), ((), ())),
                                   preferred_element_type=jnp.float32)
    msk_ref[...] = ((lab_ref[...].reshape(LBLK, 1) != 0) & hq).astype(jnp.int8)


def _tc_cosine(qsum, qidx, qidxt, le, lab3d):
    return pl.pallas_call(
        _tc_body,
        grid=(NLB,),
        in_specs=[
            pl.BlockSpec((B, DIM), lambda i: (0, 0)),
            pl.BlockSpec((B, TP), lambda i: (0, 0)),
            pl.BlockSpec((TP, B), lambda i: (0, 0)),
            pl.BlockSpec((LBLK, DIM), lambda i: (i, 0)),
            pl.BlockSpec((1, LBLK, 1), lambda i: (i, 0, 0)),
        ],
        out_specs=[
            pl.BlockSpec((LBLK, B), lambda i: (i, 0)),
            pl.BlockSpec((LBLK, B), lambda i: (i, 0)),
        ],
        out_shape=[jax.ShapeDtypeStruct((LF, B), jnp.float32),
                   jax.ShapeDtypeStruct((LF, B), jnp.int8)],
    )(qsum, qidx, qidxt, le, lab3d)


def kernel(queries, labels, table):
    qpad = jnp.pad(queries, ((0, 0), (0, TP - T)))
    lflat = jnp.pad(labels.T.reshape(-1), (0, LPAD - LF))
    pmat = jnp.eye(DIM, 2 * DIM, dtype=jnp.float32)
    tablep = lax.dot_general(table, pmat, (((1,), (0,)), ((), ())),
                             preferred_element_type=jnp.float32)
    tablep = tablep.reshape(2 * VOCAB, DIM)
    qsum, le = _sc_gather()(qpad.reshape(-1) * 2, lflat * 2, tablep)
    simt, maskt = _tc_cosine(qsum, qpad, qpad.T, le,
                             (lflat != 0).astype(jnp.int8).reshape(NLB, LBLK, 1))
    sim = simt.reshape(1, W, L, B).transpose(0, 3, 2, 1)
    msk = maskt.view(jnp.bool_)
    msk = msk.reshape(1, W, L, B).transpose(0, 3, 2, 1)
    return (sim, msk)


# speedup vs baseline: 1.0278x; 1.0011x over previous
import functools

import jax
import jax.numpy as jnp
from jax import lax
from jax.experimental import pallas as pl
from jax.experimental.pallas import tpu as pltpu
from jax.experimental.pallas import tpu_sc as plsc

VOCAB = 1000000
DIM = 64
B, T = 1024, 50
L, W = 1000, 10
LF = L * W
LPAD = 10240
TP = 56
EPS = 1e-8

NC, NS = 2, 16
NW = NC * NS
QPW = B // NW
LPW = LPAD // NW
LCH = 64
NBUF = 6


def _sc_body(qidx_hbm, lidx_hbm, table_hbm, qsum_hbm, le_hbm,
             qidx_v, lidx_v, qblk, lbuf, rows, qsems, lsem):
    c = lax.axis_index("c")
    s = lax.axis_index("s")
    wid = c * NS + s

    pltpu.sync_copy(qidx_hbm.at[pl.ds(wid * (QPW * TP), QPW * TP)], qidx_v)
    pltpu.sync_copy(lidx_hbm.at[pl.ds(wid * LPW, LPW)], lidx_v)

    for ci in range(LPW // LCH):
        pltpu.make_async_copy(
            table_hbm.at[lidx_v.at[pl.ds(ci * LCH, LCH)]],
            lbuf.at[pl.ds(ci * LCH, LCH)], lsem).start()

    def qcopy(q, b):
        return pltpu.make_async_copy(
            table_hbm.at[qidx_v.at[pl.ds(q * TP, T)]], rows[b], qsems[b])

    def accum(q, b):
        def rbody(r, accs):
            accs = tuple(a + rows[b][2 * r, pl.ds(16 * d, 16)]
                         for d, a in enumerate(accs))
            return tuple(a + rows[b][2 * r + 1, pl.ds(16 * d, 16)]
                         for d, a in enumerate(accs))
        accs = lax.fori_loop(
            0, T // 2, rbody,
            tuple(jnp.zeros((16,), jnp.float32) for _ in range(4)))
        for d in range(4):
            qblk[q, pl.ds(16 * d, 16)] = accs[d]

    for b in range(NBUF):
        qcopy(b, b).start()
    for q in range(QPW):
        b = q % NBUF
        qcopy(q, b).wait()
        accum(q, b)
        if q + NBUF < QPW:
            qcopy(q + NBUF, b).start()

    pltpu.sync_copy(qblk, qsum_hbm.at[pl.ds(wid * QPW, QPW)])

    for ci in range(LPW // LCH):
        pltpu.make_async_copy(
            table_hbm.at[lidx_v.at[pl.ds(ci * LCH, LCH)]],
            lbuf.at[pl.ds(ci * LCH, LCH)], lsem).wait()
    pltpu.sync_copy(lbuf, le_hbm.at[pl.ds(wid * LPW, LPW)])


@functools.cache
def _sc_gather():
    return pl.kernel(
        _sc_body,
        out_type=[jax.ShapeDtypeStruct((B, DIM), jnp.float32),
                  jax.ShapeDtypeStruct((LPAD, DIM), jnp.float32)],
        mesh=plsc.VectorSubcoreMesh(core_axis_name="c", subcore_axis_name="s",
                                    num_cores=NC, num_subcores=NS),
        compiler_params=pltpu.CompilerParams(use_tc_tiling_on_sc=False),
        scratch_types=[
            pltpu.VMEM((QPW * TP,), jnp.int32),
            pltpu.VMEM((LPW,), jnp.int32),
            pltpu.VMEM((QPW, DIM), jnp.float32),
            pltpu.VMEM((LPW, DIM), jnp.float32),
            [pltpu.VMEM((T, DIM), jnp.float32) for _ in range(NBUF)],
            [pltpu.SemaphoreType.DMA for _ in range(NBUF)],
            pltpu.SemaphoreType.DMA,
        ],
    )


LBLK = 2048
NLB = LPAD // LBLK


def _tc_body(qsum_ref, qidx_ref, qidxt_ref, le_ref, lab_ref, sim_ref, msk_ref):
    nw = jnp.sum((qidx_ref[...] != 0).astype(jnp.float32), axis=1,
                 keepdims=True)
    qe = qsum_ref[...] / nw
    na = jnp.maximum(jnp.sqrt(jnp.sum(qe * qe, axis=1, keepdims=True)), EPS)
    qn = qe / na
    hq = jnp.sum((qidxt_ref[...] != 0).astype(jnp.int32), axis=0,
                 keepdims=True) > 0
    le = le_ref[...]
    nb = jnp.maximum(jnp.sqrt(jnp.sum(le * le, axis=1, keepdims=True)), EPS)
    ln = le / nb
    sim_ref[...] = lax.dot_general(ln, qn, (((1,), (1,)), ((), ())),
                                   preferred_element_type=jnp.float32)
    msk_ref[...] = ((lab_ref[...].reshape(LBLK, 1) != 0) & hq).astype(jnp.int8)


def _tc_cosine(qsum, qidx, qidxt, le, lab3d):
    return pl.pallas_call(
        _tc_body,
        grid=(NLB,),
        in_specs=[
            pl.BlockSpec((B, DIM), lambda i: (0, 0)),
            pl.BlockSpec((B, TP), lambda i: (0, 0)),
            pl.BlockSpec((TP, B), lambda i: (0, 0)),
            pl.BlockSpec((LBLK, DIM), lambda i: (i, 0)),
            pl.BlockSpec((1, LBLK, 1), lambda i: (i, 0, 0)),
        ],
        out_specs=[
            pl.BlockSpec((LBLK, B), lambda i: (i, 0)),
            pl.BlockSpec((LBLK, B), lambda i: (i, 0)),
        ],
        out_shape=[jax.ShapeDtypeStruct((LF, B), jnp.float32),
                   jax.ShapeDtypeStruct((LF, B), jnp.int8)],
    )(qsum, qidx, qidxt, le, lab3d)


def kernel(queries, labels, table):
    qpad = jnp.pad(queries, ((0, 0), (0, TP - T)))
    lflat = jnp.pad(labels.T.reshape(-1), (0, LPAD - LF))
    pmat = jnp.eye(DIM, 2 * DIM, dtype=jnp.float32)
    tablep = lax.dot_general(table, pmat, (((1,), (0,)), ((), ())),
                             preferred_element_type=jnp.float32)
    tablep = tablep.reshape(2 * VOCAB, DIM)
    qsum, le = _sc_gather()(qpad.reshape(-1) * 2, lflat * 2, tablep)
    simt, maskt = _tc_cosine(qsum, qpad, qpad.T, le,
                             (lflat != 0).astype(jnp.int8).reshape(NLB, LBLK, 1))
    sim = simt.reshape(1, W, L, B).transpose(0, 3, 2, 1)
    msk = maskt.view(jnp.bool_)
    msk = msk.reshape(1, W, L, B).transpose(0, 3, 2, 1)
    return (sim, msk)
